# baseline (speedup 1.0000x reference)
import functools

import jax
import jax.numpy as jnp
from jax import lax
from jax.experimental import pallas as pl
from jax.experimental.pallas import tpu as pltpu
from jax.experimental.pallas import tpu_sc as plsc

B = 16384
N_DENSE = 13
N_SPARSE = 26
VOCAB = 100000
EMB_DIM = 16

NC = 2
NS = 16
NW = NC * NS

BPT = B // NW
RPT = BPT * N_SPARSE
TROW = N_SPARSE * VOCAB // 8
CB = 4
CROWS = CB * N_SPARSE
NCHUNK = BPT // CB
NBUF = 4
OFF_PERIOD = 208
OFF_VECS = OFF_PERIOD // 16


def _sc_body(tbl_hbm, idx_hbm, offs_hbm, dv_hbm, dw_hbm, out_hbm,
             idx_v, row8_v, offs_v, dv_v, dw_v, out_v, acc_v,
             buf0, buf1, buf2, buf3, sem0, sem1, sem2, sem3):
    bufs = (buf0, buf1, buf2, buf3)
    sems = (sem0, sem1, sem2, sem3)
    wid = lax.axis_index("s") * NC + lax.axis_index("c")
    ibase = wid * RPT
    bbase = wid * BPT

    pltpu.sync_copy(idx_hbm.at[pl.ds(ibase, RPT)], idx_v.at[pl.ds(0, RPT)])
    pltpu.sync_copy(offs_hbm, offs_v)
    pltpu.sync_copy(dw_hbm, dw_v)
    pltpu.sync_copy(dv_hbm.at[pl.ds(bbase * EMB_DIM, BPT * EMB_DIM)], dv_v)

    def off_body(o, carry):
        for j in range(OFF_VECS):
            sl = pl.ds((o * OFF_VECS + j) * 16, 16)
            f = idx_v[sl] + offs_v[pl.ds(j * 16, 16)]
            idx_v[sl] = (f & 7) * 16
            row8_v[sl] = lax.shift_right_logical(f, 3)
        return carry

    lax.fori_loop(0, RPT // OFF_PERIOD, off_body, 0)

    def fire(chunk, slot):
        pltpu.make_async_copy(
            tbl_hbm.at[row8_v.at[pl.ds(chunk * CROWS, CROWS)]],
            bufs[slot], sems[slot]).start()

    def drain(chunk, slot):
        pltpu.make_async_copy(
            tbl_hbm.at[row8_v.at[pl.ds(chunk * CROWS, CROWS)]],
            bufs[slot], sems[slot]).wait()

    for s in range(NBUF):
        fire(s, s)

    dwv = dw_v[...]
    lanes16 = lax.iota(jnp.int32, 16) * 16

    def chunk_body(g, carry):
        for s in range(NBUF):
            c = g * NBUF + s
            drain(c, s)
            buf = bufs[s]
            for j in range(CB):
                bl = c * CB + j
                q0 = c * CROWS + j * N_SPARSE
                st0 = idx_v[pl.ds(q0, 16)]
                st1 = idx_v[pl.ds(q0 + 16, 16)]
                acc = dv_v[pl.ds(bl * EMB_DIM, EMB_DIM)] * dwv
                for r in range(N_SPARSE):
                    p = j * N_SPARSE + r
                    st = st0[r] if r < 16 else st1[r - 16]
                    acc = acc + buf[p, pl.ds(st, EMB_DIM)]
                acc_v[pl.ds((s * CB + j) * 16, 16)] = acc

            @pl.when(g < NCHUNK // NBUF - 1)
            def _():
                fire(c + NBUF, s)

        r0 = plsc.load_gather(acc_v, [lanes16 + 0])
        r1 = plsc.load_gather(acc_v, [lanes16 + 1])
        r2 = plsc.load_gather(acc_v, [lanes16 + 2])
        r3 = plsc.load_gather(acc_v, [lanes16 + 3])
        for d in range(4, 16, 4):
            r0 = r0 + plsc.load_gather(acc_v, [lanes16 + d])
            r1 = r1 + plsc.load_gather(acc_v, [lanes16 + d + 1])
            r2 = r2 + plsc.load_gather(acc_v, [lanes16 + d + 2])
            r3 = r3 + plsc.load_gather(acc_v, [lanes16 + d + 3])
        out_v[pl.ds(g * 16, 16)] = (r0 + r1) + (r2 + r3)
        return carry

    lax.fori_loop(0, NCHUNK // NBUF, chunk_body, 0)

    pltpu.sync_copy(out_v, out_hbm.at[pl.ds(bbase, BPT)])


@functools.partial(
    pl.kernel,
    out_type=jax.ShapeDtypeStruct((B,), jnp.float32),
    mesh=plsc.VectorSubcoreMesh(core_axis_name="c", subcore_axis_name="s"),
    compiler_params=pltpu.CompilerParams(needs_layout_passes=False),
    scratch_types=[
        pltpu.VMEM((RPT + 16,), jnp.int32),
        pltpu.VMEM((RPT,), jnp.int32),
        pltpu.VMEM((OFF_PERIOD,), jnp.int32),
        pltpu.VMEM((BPT * EMB_DIM,), jnp.float32),
        pltpu.VMEM((EMB_DIM,), jnp.float32),
        pltpu.VMEM((BPT,), jnp.float32),
        pltpu.VMEM((256,), jnp.float32),
        pltpu.VMEM((CROWS, 128), jnp.float32),
        pltpu.VMEM((CROWS, 128), jnp.float32),
        pltpu.VMEM((CROWS, 128), jnp.float32),
        pltpu.VMEM((CROWS, 128), jnp.float32),
        pltpu.SemaphoreType.DMA,
        pltpu.SemaphoreType.DMA,
        pltpu.SemaphoreType.DMA,
        pltpu.SemaphoreType.DMA,
    ],
)
def _sc_linear(tbl_hbm, idx_hbm, offs_hbm, dv_hbm, dw_hbm, out_hbm,
               idx_v, row8_v, offs_v, dv_v, dw_v, out_v, acc_v,
               buf0, buf1, buf2, buf3, sem0, sem1, sem2, sem3):
    _sc_body(tbl_hbm, idx_hbm, offs_hbm, dv_hbm, dw_hbm, out_hbm,
             idx_v, row8_v, offs_v, dv_v, dw_v, out_v, acc_v,
             buf0, buf1, buf2, buf3, sem0, sem1, sem2, sem3)


def kernel(inputs, emb_tables, dense_weight, bias):
    idx = inputs[:, N_DENSE:N_DENSE + N_SPARSE].astype(jnp.int32).reshape(-1)
    tbl = emb_tables.reshape(TROW, 128)
    offs = ((jnp.arange(OFF_PERIOD, dtype=jnp.int32) % N_SPARSE)
            * jnp.int32(VOCAB))
    dv = jnp.concatenate(
        [inputs[:, :N_DENSE],
         jnp.ones((B, 1), jnp.float32),
         jnp.zeros((B, EMB_DIM - N_DENSE - 1), jnp.float32)], axis=1
    ).reshape(-1)
    dw = jnp.concatenate(
        [dense_weight[:, 0], bias,
         jnp.zeros((EMB_DIM - N_DENSE - 1,), jnp.float32)])
    out = _sc_linear(tbl, idx, offs, dv, dw)
    return out.reshape(B, 1)


# speedup vs baseline: 4.3496x; 4.3496x over previous
import functools

import jax
import jax.numpy as jnp
from jax import lax
from jax.experimental import pallas as pl
from jax.experimental.pallas import tpu as pltpu
from jax.experimental.pallas import tpu_sc as plsc

B = 16384
N_DENSE = 13
N_SPARSE = 26
VOCAB = 100000
EMB_DIM = 16

NC = 2
NS = 16
NW = NC * NS

BPT = B // NW
RPT = BPT * N_SPARSE
NROWS = N_SPARSE * VOCAB
TN = 104
NT = RPT // TN
SGB = 16
NSG = BPT // SGB
OFF_PERIOD = 208
OFF_VECS = OFF_PERIOD // 16


def _sc_body(rs_hbm, idx_hbm, offs_hbm, dvt_hbm, dwt_hbm, out_hbm,
             idx_v, offs_v, dvt_v, dwt_v, out_v, val_v, sem):
    wid = lax.axis_index("s") * NC + lax.axis_index("c")
    ibase = wid * RPT
    bbase = wid * BPT

    pltpu.sync_copy(idx_hbm.at[pl.ds(ibase, RPT)], idx_v)
    pltpu.sync_copy(offs_hbm, offs_v)
    pltpu.sync_copy(dwt_hbm, dwt_v)
    pltpu.sync_copy(dvt_hbm.at[:, pl.ds(bbase, BPT)], dvt_v)

    def off_body(o, carry):
        for j in range(OFF_VECS):
            sl = pl.ds((o * OFF_VECS + j) * 16, 16)
            idx_v[sl] = idx_v[sl] + offs_v[pl.ds(j * 16, 16)]
        return carry

    lax.fori_loop(0, RPT // OFF_PERIOD, off_body, 0)

    def fire_body(t, carry):
        pltpu.make_async_copy(
            rs_hbm.at[idx_v.at[pl.ds(t * TN, TN)]],
            val_v.at[pl.ds(t * TN, TN)], sem).start()
        return carry

    lax.fori_loop(0, NT, fire_body, 0)

    pltpu.make_async_copy(rs_hbm.at[pl.ds(0, RPT)], val_v, sem).wait()

    lanes26 = lax.iota(jnp.int32, 16) * N_SPARSE

    def sg_body(sg, carry):
        base = lanes26 + sg * (SGB * N_SPARSE)
        b0 = plsc.load_gather(val_v, [base + 0])
        b1 = plsc.load_gather(val_v, [base + 1])
        b2 = dvt_v[0, pl.ds(sg * SGB, 16)] * dwt_v[0, :]
        b3 = dvt_v[1, pl.ds(sg * SGB, 16)] * dwt_v[1, :]
        for f in range(2, N_SPARSE, 2):
            b0 = b0 + plsc.load_gather(val_v, [base + f])
            b1 = b1 + plsc.load_gather(val_v, [base + f + 1])
        for k in range(2, EMB_DIM, 2):
            b2 = b2 + dvt_v[k, pl.ds(sg * SGB, 16)] * dwt_v[k, :]
            b3 = b3 + dvt_v[k + 1, pl.ds(sg * SGB, 16)] * dwt_v[k + 1, :]
        out_v[pl.ds(sg * SGB, 16)] = (b0 + b1) + (b2 + b3)
        return carry

    lax.fori_loop(0, NSG, sg_body, 0)

    pltpu.sync_copy(out_v, out_hbm.at[pl.ds(bbase, BPT)])


@functools.partial(
    pl.kernel,
    out_type=jax.ShapeDtypeStruct((B,), jnp.float32),
    mesh=plsc.VectorSubcoreMesh(core_axis_name="c", subcore_axis_name="s"),
    compiler_params=pltpu.CompilerParams(needs_layout_passes=False),
    scratch_types=[
        pltpu.VMEM((RPT,), jnp.int32),
        pltpu.VMEM((OFF_PERIOD,), jnp.int32),
        pltpu.VMEM((EMB_DIM, BPT), jnp.float32),
        pltpu.VMEM((EMB_DIM, EMB_DIM), jnp.float32),
        pltpu.VMEM((BPT,), jnp.float32),
        pltpu.VMEM((RPT,), jnp.float32),
        pltpu.SemaphoreType.DMA,
    ],
)
def _sc_linear(rs_hbm, idx_hbm, offs_hbm, dvt_hbm, dwt_hbm, out_hbm,
               idx_v, offs_v, dvt_v, dwt_v, out_v, val_v, sem):
    _sc_body(rs_hbm, idx_hbm, offs_hbm, dvt_hbm, dwt_hbm, out_hbm,
             idx_v, offs_v, dvt_v, dwt_v, out_v, val_v, sem)


def kernel(inputs, emb_tables, dense_weight, bias):
    idx = inputs[:, N_DENSE:N_DENSE + N_SPARSE].astype(jnp.int32).reshape(-1)
    rowsum = jnp.sum(emb_tables, axis=2).reshape(NROWS)
    offs = ((jnp.arange(OFF_PERIOD, dtype=jnp.int32) % N_SPARSE)
            * jnp.int32(VOCAB))
    dvt = jnp.concatenate(
        [inputs[:, :N_DENSE],
         jnp.ones((B, 1), jnp.float32),
         jnp.zeros((B, EMB_DIM - N_DENSE - 1), jnp.float32)], axis=1).T
    dwt = jnp.tile(
        jnp.concatenate([dense_weight[:, 0], bias,
                         jnp.zeros((EMB_DIM - N_DENSE - 1,), jnp.float32)]
                        )[:, None], (1, EMB_DIM))
    out = _sc_linear(rowsum, idx, offs, dvt, dwt)
    return out.reshape(B, 1)
